# Initial kernel scaffold; baseline (speedup 1.0000x reference)
import jax
import jax.numpy as jnp
from jax.experimental import pallas as pl


def kernel(features, targets, cams, epoch, all_pseudo_label, all_proxy_label, global_memory):
    raise NotImplementedError("write your pallas kernel here")


# speedup vs baseline: 30.5801x; 30.5801x over previous
import dataclasses
import functools

import jax
import jax.numpy as jnp
from jax import lax
from jax.experimental import pallas as pl
from jax.experimental.pallas import tpu as pltpu
from jax.experimental.pallas import tpu_sc as plsc

TEMP = 0.05
BG_KNN = 50
POSK = 3
BAL_W = 0.15
NUM_CAMS = 8
M = 32768
D = 256
B = 64
PPL = 4
LANES = 128
GD = M // LANES
MBLK = 2048
NBLK = M // MBLK
GPB = MBLK // LANES
NEG = -1e30
BISECT_ITERS = 36


def _sc_gather(targets, apl_rows, memory):
    mesh = plsc.VectorSubcoreMesh(core_axis_name="c", subcore_axis_name="s")
    cp = pltpu.CompilerParams()
    if "needs_layout_passes" in pltpu.CompilerParams.__dataclass_fields__:
        cp = dataclasses.replace(cp, needs_layout_passes=False)

    @functools.partial(
        pl.kernel,
        mesh=mesh,
        compiler_params=cp,
        out_type=[
            jax.ShapeDtypeStruct((B,), jnp.int32),
            jax.ShapeDtypeStruct((B, D), jnp.float32),
        ],
        scratch_types=[
            pltpu.VMEM((B,), jnp.int32),
            pltpu.VMEM((B,), jnp.int32),
            pltpu.VMEM((B, 128), jnp.int32),
            pltpu.VMEM((B,), jnp.int32),
            pltpu.VMEM((B, D), jnp.float32),
            pltpu.SemaphoreType.DMA,
        ],
    )
    def kern(tgt_hbm, apl_hbm, mem_hbm, pt_out, g_out,
             tgt_v, row_v, lab_v, pt_v, g_v, sem):
        wid = lax.axis_index("c") * 16 + lax.axis_index("s")

        @pl.when(wid == 0)
        def _():
            pltpu.sync_copy(tgt_hbm, tgt_v)
            for k in range(B // 16):
                t = tgt_v[pl.ds(16 * k, 16)]
                row_v[pl.ds(16 * k, 16)] = lax.shift_right_logical(t, 7)
            pltpu.async_copy(apl_hbm.at[row_v], lab_v, sem).wait()
            for k in range(B // 16):
                t = tgt_v[pl.ds(16 * k, 16)]
                col = lax.bitwise_and(t, 127)
                rowi = lax.iota(jnp.int32, 16) + 16 * k
                pt_v[pl.ds(16 * k, 16)] = plsc.load_gather(lab_v, [rowi, col])
            pltpu.sync_copy(pt_v, pt_out)
            pltpu.async_copy(mem_hbm.at[pt_v], g_v, sem).wait()
            pltpu.sync_copy(g_v, g_out)

    return kern(targets, apl_rows, memory)


def _dot_t(a, b):
    return lax.dot_general(a, b, (((1,), (1,)), ((), ())),
                           precision=lax.Precision.HIGHEST,
                           preferred_element_type=jnp.float32)


def _sum2(a):
    return jnp.sum(jnp.sum(a, axis=2), axis=1)


def _max2(a):
    return jnp.max(jnp.max(a, axis=2), axis=1)


def _bisect(w_ref, lo, hi, k_count):
    def body(_, carry):
        lo, hi = carry
        mid = 0.5 * (lo + hi)
        cnt = _sum2((w_ref[...] > mid[:, None, None]).astype(jnp.float32))
        ge = cnt >= k_count
        return jnp.where(ge, mid, lo), jnp.where(ge, hi, mid)

    lo, hi = lax.fori_loop(0, BISECT_ITERS, body, (lo, hi))
    return hi


def _tc_body(feat_ref, g_ref, pt_ref, cam_ref, mem_ref, out_ref, x3, s3, w3):
    b = pl.program_id(0)

    @pl.when(b < NBLK)
    def _():
        blk = mem_ref[...]
        sc = _dot_t(feat_ref[...], blk)
        sr = _dot_t(g_ref[...], blk)
        x3[:, pl.ds(b * GPB, GPB), :] = (sc / TEMP).reshape(B, GPB, LANES)
        s3[:, pl.ds(b * GPB, GPB), :] = (
            BAL_W * sc + (1.0 - BAL_W) * sr).reshape(B, GPB, LANES)

    @pl.when(b == NBLK)
    def _():
        x = x3[...]
        s = s3[...]
        cam = cam_ref[...]
        pt = pt_ref[...]
        lbl = lax.shift_right_logical(pt, 2)

        gi = lax.broadcasted_iota(jnp.int32, (B, GD, LANES), 1)
        li = lax.broadcasted_iota(jnp.int32, (B, GD, LANES), 2)
        j3 = gi * LANES + li

        x_at_t = _sum2(jnp.where(j3 == pt[:, :, None], x, 0.0))

        lane8 = lax.bitwise_and(
            lax.broadcasted_iota(jnp.int32, (B, LANES), 1), 7)
        lmask = lane8 == cam
        mx_lane = jnp.max(x, axis=1)
        m_int = jnp.max(jnp.where(lmask, mx_lane, NEG), axis=1)
        e_int = jnp.where(lmask[:, None, :],
                          jnp.exp(x - m_int[:, None, None]), 0.0)
        s_int = _sum2(e_int)
        t_in_cam = lax.bitwise_and(pt[:, 0], 7) == cam[:, 0]
        intra = jnp.where(t_in_cam, m_int + jnp.log(s_int) - x_at_t, 0.0)

        pmask = lax.shift_right_logical(j3, 2) == lbl[:, :, None]
        pos_sum = _sum2(jnp.where(pmask, x, 0.0))
        pos_max = _max2(jnp.where(pmask, x, NEG))
        w3[...] = jnp.where(pmask, NEG, x)
        m_row = _max2(w3[...])
        v_a = _bisect(w3, jnp.full((B,), -21.0, jnp.float32), m_row,
                      float(BG_KNN))
        xm = w3[...]
        gt_a = xm > v_a[:, None, None]
        cnt_gt = _sum2(gt_a.astype(jnp.float32))
        m_a = jnp.maximum(m_row, pos_max)
        sum_a = (_sum2(jnp.where(gt_a, jnp.exp(xm - m_a[:, None, None]), 0.0))
                 + (BG_KNN - cnt_gt) * jnp.exp(v_a - m_a)
                 + _sum2(jnp.where(pmask,
                                   jnp.exp(x - m_a[:, None, None]), 0.0)))
        assoc = m_a + jnp.log(sum_a) - 0.25 * pos_sum

        ms_lane = jnp.max(s, axis=1)
        g_at = jnp.min(jnp.where(s == ms_lane[:, None, :], gi, M), axis=1)
        ip_at = jnp.sum(jnp.where(gi == g_at[:, None, :], x, 0.0), axis=1)
        li2 = lax.broadcasted_iota(jnp.int32, (B, LANES), 1)
        j_at = g_at * LANES + li2

        mc_l, jc_l, ic_l = [], [], []
        for c in range(NUM_CAMS):
            mk = lane8 == c
            mc = jnp.max(jnp.where(mk, ms_lane, NEG), axis=1)
            ln = jnp.min(jnp.where(mk & (ms_lane == mc[:, None]),
                                   li2, LANES), axis=1)
            onel = li2 == ln[:, None]
            jc_l.append(jnp.sum(jnp.where(onel, j_at, 0), axis=1)[:, None])
            ic_l.append(jnp.sum(jnp.where(onel, ip_at, 0.0), axis=1)[:, None])
            mc_l.append(mc[:, None])
        m8 = jnp.concatenate(mc_l, axis=1)
        j8 = jnp.concatenate(jc_l, axis=1)
        i8 = jnp.concatenate(ic_l, axis=1)
        c8 = lax.broadcasted_iota(jnp.int32, (B, NUM_CAMS), 1)

        ch_j, ch_i = [], []
        for _k in range(POSK):
            vmax = jnp.max(m8, axis=1)
            cw = jnp.min(jnp.where(m8 == vmax[:, None], c8, NUM_CAMS), axis=1)
            sel = c8 == cw[:, None]
            ch_j.append(jnp.sum(jnp.where(sel, j8, 0), axis=1))
            ch_i.append(jnp.sum(jnp.where(sel, i8, 0.0), axis=1))
            m8 = jnp.where(sel, NEG, m8)

        chm = ((j3 == ch_j[0][:, None, None])
               | (j3 == ch_j[1][:, None, None])
               | (j3 == ch_j[2][:, None, None]))
        w3[...] = jnp.where(chm, NEG, s)
        m2_row = _max2(w3[...])
        v_o = _bisect(w3, jnp.full((B,), -1.5, jnp.float32), m2_row,
                      float(BG_KNN))
        sm = w3[...]
        selm = sm >= v_o[:, None, None]
        mi_sel = _max2(jnp.where(selm, x, NEG))
        ch_imax = jnp.maximum(jnp.maximum(ch_i[0], ch_i[1]), ch_i[2])
        m_o = jnp.maximum(mi_sel, ch_imax)
        sum_o = _sum2(jnp.where(selm, jnp.exp(x - m_o[:, None, None]), 0.0))
        for k in range(POSK):
            sum_o = sum_o + jnp.exp(ch_i[k] - m_o)
        online = (m_o + jnp.log(sum_o)
                  - (ch_i[0] + ch_i[1] + ch_i[2]) * (1.0 / POSK))

        camv = cam[:, 0]
        loss = jnp.float32(0.0)
        for c in range(NUM_CAMS):
            mk = camv == c
            cnt = jnp.sum(mk.astype(jnp.float32))
            ok = cnt > 0.0
            inv = 1.0 / jnp.maximum(cnt, 1.0)
            loss = loss + jnp.where(
                ok, jnp.sum(jnp.where(mk, intra, 0.0)) * inv, 0.0)
            loss = loss + jnp.where(
                ok, jnp.sum(jnp.where(mk, assoc, 0.0)) * inv, 0.0)
            loss = loss + jnp.where(
                ok, jnp.sum(jnp.where(mk, online, 0.0)) * inv, 0.0)
        out_ref[...] = jnp.reshape(loss, (1, 1))


def _tc_main(features, g_rows, pt, cams, memory, interpret=False):
    return pl.pallas_call(
        _tc_body,
        grid=(NBLK + 1,),
        in_specs=[
            pl.BlockSpec((B, D), lambda b: (0, 0)),
            pl.BlockSpec((B, D), lambda b: (0, 0)),
            pl.BlockSpec((B, 1), lambda b: (0, 0)),
            pl.BlockSpec((B, 1), lambda b: (0, 0)),
            pl.BlockSpec((MBLK, D), lambda b: (jnp.minimum(b, NBLK - 1), 0)),
        ],
        out_specs=pl.BlockSpec((1, 1), lambda b: (0, 0)),
        out_shape=jax.ShapeDtypeStruct((1, 1), jnp.float32),
        scratch_shapes=[
            pltpu.VMEM((B, GD, LANES), jnp.float32),
            pltpu.VMEM((B, GD, LANES), jnp.float32),
            pltpu.VMEM((B, GD, LANES), jnp.float32),
        ],
        compiler_params=pltpu.CompilerParams(
            dimension_semantics=("arbitrary",)),
        interpret=interpret,
    )(features, g_rows, pt, cams, memory)


def kernel(features, targets, cams, epoch, all_pseudo_label, all_proxy_label,
           global_memory):
    del epoch, all_pseudo_label
    apl_rows = all_proxy_label.astype(jnp.int32).reshape(-1, 128)
    pt, g_rows = _sc_gather(targets.astype(jnp.int32), apl_rows,
                            global_memory)
    out = _tc_main(features, g_rows, pt.reshape(B, 1),
                   cams.astype(jnp.int32).reshape(B, 1), global_memory)
    return out[0, 0]


# speedup vs baseline: 59.1236x; 1.9334x over previous
import dataclasses
import functools

import jax
import jax.numpy as jnp
from jax import lax
from jax.experimental import pallas as pl
from jax.experimental.pallas import tpu as pltpu
from jax.experimental.pallas import tpu_sc as plsc

TEMP = 0.05
BG_KNN = 50
POSK = 3
BAL_W = 0.15
NUM_CAMS = 8
M = 32768
D = 256
B = 64
PPL = 4
LANES = 128
GD = M // LANES
MBLK = 2048
NBLK = M // MBLK
GPB = MBLK // LANES
NEG = -1e30
BISECT_ITERS = 36


def _sc_gather(targets, apl_rows, memory):
    mesh = plsc.VectorSubcoreMesh(core_axis_name="c", subcore_axis_name="s")
    cp = pltpu.CompilerParams()
    if "needs_layout_passes" in pltpu.CompilerParams.__dataclass_fields__:
        cp = dataclasses.replace(cp, needs_layout_passes=False)

    @functools.partial(
        pl.kernel,
        mesh=mesh,
        compiler_params=cp,
        out_type=[
            jax.ShapeDtypeStruct((B,), jnp.int32),
            jax.ShapeDtypeStruct((B, D), jnp.float32),
        ],
        scratch_types=[
            pltpu.VMEM((B,), jnp.int32),
            pltpu.VMEM((B,), jnp.int32),
            pltpu.VMEM((B, 128), jnp.int32),
            pltpu.VMEM((B,), jnp.int32),
            pltpu.VMEM((B, D), jnp.float32),
            pltpu.SemaphoreType.DMA,
        ],
    )
    def kern(tgt_hbm, apl_hbm, mem_hbm, pt_out, g_out,
             tgt_v, row_v, lab_v, pt_v, g_v, sem):
        wid = lax.axis_index("c") * 16 + lax.axis_index("s")

        @pl.when(wid == 0)
        def _():
            pltpu.sync_copy(tgt_hbm, tgt_v)
            for k in range(B // 16):
                t = tgt_v[pl.ds(16 * k, 16)]
                row_v[pl.ds(16 * k, 16)] = lax.shift_right_logical(t, 7)
            pltpu.async_copy(apl_hbm.at[row_v], lab_v, sem).wait()
            for k in range(B // 16):
                t = tgt_v[pl.ds(16 * k, 16)]
                col = lax.bitwise_and(t, 127)
                rowi = lax.iota(jnp.int32, 16) + 16 * k
                pt_v[pl.ds(16 * k, 16)] = plsc.load_gather(lab_v, [rowi, col])
            pltpu.sync_copy(pt_v, pt_out)
            pltpu.async_copy(mem_hbm.at[pt_v], g_v, sem).wait()
            pltpu.sync_copy(g_v, g_out)

    return kern(targets, apl_rows, memory)


def _dot_t(a, b):
    return lax.dot_general(a, b, (((1,), (1,)), ((), ())),
                           precision=lax.Precision.HIGHEST,
                           preferred_element_type=jnp.float32)


def _sum2(a):
    return jnp.sum(jnp.sum(a, axis=1), axis=1)


def _max2(a):
    return jnp.max(jnp.max(a, axis=1), axis=1)


def _bisect(w_ref, lo, hi, k_count, iters):
    def body(_, carry):
        lo, hi = carry
        mid = 0.5 * (lo + hi)
        cnt = _sum2((w_ref[...] > mid[:, None, None]).astype(jnp.float32))
        ge = cnt >= k_count
        return jnp.where(ge, mid, lo), jnp.where(ge, hi, mid)

    lo, hi = lax.fori_loop(0, iters, body, (lo, hi))
    return hi


def _tc_body(feat_ref, g_ref, pt_ref, cam_ref, mem_ref, out_ref, x3, s3, w3):
    b = pl.program_id(0)

    @pl.when(b < NBLK)
    def _():
        blk = mem_ref[...]
        sc = _dot_t(feat_ref[...], blk)
        sr = _dot_t(g_ref[...], blk)
        x3[:, pl.ds(b * GPB, GPB), :] = (sc / TEMP).reshape(B, GPB, LANES)
        s3[:, pl.ds(b * GPB, GPB), :] = (
            BAL_W * sc + (1.0 - BAL_W) * sr).reshape(B, GPB, LANES)

    @pl.when(b == NBLK)
    def _():
        x = x3[...]
        s = s3[...]
        cam = cam_ref[...]
        pt = pt_ref[...]
        lbl = lax.shift_right_logical(pt, 2)

        gi = lax.broadcasted_iota(jnp.int32, (B, GD, LANES), 1)
        li = lax.broadcasted_iota(jnp.int32, (B, GD, LANES), 2)
        j3 = gi * LANES + li

        x_at_t = _sum2(jnp.where(j3 == pt[:, :, None], x, 0.0))

        lane8 = lax.bitwise_and(
            lax.broadcasted_iota(jnp.int32, (B, LANES), 1), 7)
        lmask = lane8 == cam
        mx_lane = jnp.max(x, axis=1)
        m_int = jnp.max(jnp.where(lmask, mx_lane, NEG), axis=1)
        e_int = jnp.where(lmask[:, None, :],
                          jnp.exp(x - m_int[:, None, None]), 0.0)
        s_int = _sum2(e_int)
        t_in_cam = lax.bitwise_and(pt[:, 0], 7) == cam[:, 0]
        intra = jnp.where(t_in_cam, m_int + jnp.log(s_int) - x_at_t, 0.0)

        pmask = lax.shift_right_logical(j3, 2) == lbl[:, :, None]
        pos_sum = _sum2(jnp.where(pmask, x, 0.0))
        pos_max = _max2(jnp.where(pmask, x, NEG))
        w3[...] = jnp.where(pmask, NEG, x)
        m_row = _max2(w3[...])
        v_a = _bisect(w3, jnp.full((B,), -21.0, jnp.float32), m_row,
                      float(BG_KNN), 16)
        xm = w3[...]
        gt_a = xm > v_a[:, None, None]
        cnt_gt = _sum2(gt_a.astype(jnp.float32))
        m_a = jnp.maximum(m_row, pos_max)
        sum_a = (_sum2(jnp.where(gt_a, jnp.exp(xm - m_a[:, None, None]), 0.0))
                 + (BG_KNN - cnt_gt) * jnp.exp(v_a - m_a)
                 + _sum2(jnp.where(pmask,
                                   jnp.exp(x - m_a[:, None, None]), 0.0)))
        assoc = m_a + jnp.log(sum_a) - 0.25 * pos_sum

        ms_lane = jnp.max(s, axis=1)
        g_at = jnp.min(jnp.where(s == ms_lane[:, None, :], gi, M), axis=1)
        ip_at = jnp.sum(jnp.where(gi == g_at[:, None, :], x, 0.0), axis=1)
        li2 = lax.broadcasted_iota(jnp.int32, (B, LANES), 1)
        j_at = g_at * LANES + li2

        mc_l, jc_l, ic_l = [], [], []
        for c in range(NUM_CAMS):
            mk = lane8 == c
            mc = jnp.max(jnp.where(mk, ms_lane, NEG), axis=1)
            ln = jnp.min(jnp.where(mk & (ms_lane == mc[:, None]),
                                   li2, LANES), axis=1)
            onel = li2 == ln[:, None]
            jc_l.append(jnp.sum(jnp.where(onel, j_at, 0), axis=1)[:, None])
            ic_l.append(jnp.sum(jnp.where(onel, ip_at, 0.0), axis=1)[:, None])
            mc_l.append(mc[:, None])
        m8 = jnp.concatenate(mc_l, axis=1)
        j8 = jnp.concatenate(jc_l, axis=1)
        i8 = jnp.concatenate(ic_l, axis=1)
        c8 = lax.broadcasted_iota(jnp.int32, (B, NUM_CAMS), 1)

        ch_j, ch_i = [], []
        for _k in range(POSK):
            vmax = jnp.max(m8, axis=1)
            cw = jnp.min(jnp.where(m8 == vmax[:, None], c8, NUM_CAMS), axis=1)
            sel = c8 == cw[:, None]
            ch_j.append(jnp.sum(jnp.where(sel, j8, 0), axis=1))
            ch_i.append(jnp.sum(jnp.where(sel, i8, 0.0), axis=1))
            m8 = jnp.where(sel, NEG, m8)

        chm = ((j3 == ch_j[0][:, None, None])
               | (j3 == ch_j[1][:, None, None])
               | (j3 == ch_j[2][:, None, None]))
        w3[...] = jnp.where(chm, NEG, s)
        m2_row = _max2(w3[...])
        v_o = _bisect(w3, jnp.full((B,), -1.5, jnp.float32), m2_row,
                      float(BG_KNN), 30)
        sm = w3[...]
        selm = sm >= v_o[:, None, None]
        mi_sel = _max2(jnp.where(selm, x, NEG))
        ch_imax = jnp.maximum(jnp.maximum(ch_i[0], ch_i[1]), ch_i[2])
        m_o = jnp.maximum(mi_sel, ch_imax)
        sum_o = _sum2(jnp.where(selm, jnp.exp(x - m_o[:, None, None]), 0.0))
        for k in range(POSK):
            sum_o = sum_o + jnp.exp(ch_i[k] - m_o)
        online = (m_o + jnp.log(sum_o)
                  - (ch_i[0] + ch_i[1] + ch_i[2]) * (1.0 / POSK))

        camv = cam[:, 0]
        loss = jnp.float32(0.0)
        for c in range(NUM_CAMS):
            mk = camv == c
            cnt = jnp.sum(mk.astype(jnp.float32))
            ok = cnt > 0.0
            inv = 1.0 / jnp.maximum(cnt, 1.0)
            loss = loss + jnp.where(
                ok, jnp.sum(jnp.where(mk, intra, 0.0)) * inv, 0.0)
            loss = loss + jnp.where(
                ok, jnp.sum(jnp.where(mk, assoc, 0.0)) * inv, 0.0)
            loss = loss + jnp.where(
                ok, jnp.sum(jnp.where(mk, online, 0.0)) * inv, 0.0)
        out_ref[...] = jnp.reshape(loss, (1, 1))


def _tc_main(features, g_rows, pt, cams, memory, interpret=False):
    return pl.pallas_call(
        _tc_body,
        grid=(NBLK + 1,),
        in_specs=[
            pl.BlockSpec((B, D), lambda b: (0, 0)),
            pl.BlockSpec((B, D), lambda b: (0, 0)),
            pl.BlockSpec((B, 1), lambda b: (0, 0)),
            pl.BlockSpec((B, 1), lambda b: (0, 0)),
            pl.BlockSpec((MBLK, D), lambda b: (jnp.minimum(b, NBLK - 1), 0)),
        ],
        out_specs=pl.BlockSpec((1, 1), lambda b: (0, 0)),
        out_shape=jax.ShapeDtypeStruct((1, 1), jnp.float32),
        scratch_shapes=[
            pltpu.VMEM((B, GD, LANES), jnp.float32),
            pltpu.VMEM((B, GD, LANES), jnp.float32),
            pltpu.VMEM((B, GD, LANES), jnp.float32),
        ],
        compiler_params=pltpu.CompilerParams(
            dimension_semantics=("arbitrary",)),
        interpret=interpret,
    )(features, g_rows, pt, cams, memory)


def kernel(features, targets, cams, epoch, all_pseudo_label, all_proxy_label,
           global_memory):
    del epoch, all_pseudo_label
    apl_rows = all_proxy_label.astype(jnp.int32).reshape(-1, 128)
    pt, g_rows = _sc_gather(targets.astype(jnp.int32), apl_rows,
                            global_memory)
    out = _tc_main(features, g_rows, pt.reshape(B, 1),
                   cams.astype(jnp.int32).reshape(B, 1), global_memory)
    return out[0, 0]
